# baseline (speedup 1.0000x reference)
import jax
import jax.numpy as jnp
from jax import lax
from jax.experimental import pallas as pl
from jax.experimental.pallas import tpu as pltpu
from jax.experimental.pallas import tpu_sc as plsc

_K = 1024
_TN = 512
_NC = 2
_NS = 16
_NRG = 8
_NCG = 4
_CW = 64
_CH = 128


def _assign_body(feat_ref, sums8_ref, counts_in_ref, labels_ref, counts_ref,
                 cent_v, counts_acc):
    i = pl.program_id(0)
    nt = pl.num_programs(0)
    ft = feat_ref[...]

    @pl.when(i == 0)
    def _update_centroids():
        s8 = sums8_ref[...]
        sp = (((s8[0] + s8[1]) + (s8[2] + s8[3])) +
              ((s8[4] + s8[5]) + (s8[6] + s8[7])))
        sums = jnp.concatenate([sp[j] for j in range(_NCG)], axis=1)
        counts = counts_in_ref[:, 0:1]
        cent_v[...] = jnp.where(counts > 0.0,
                                sums / jnp.maximum(counts, 1.0), 0.0)
        counts_acc[...] = jnp.zeros_like(counts_acc)

    c = cent_v[...]
    tn, d = ft.shape
    k = c.shape[0]
    rown = jnp.sum(ft * ft, axis=1, keepdims=True)
    coln = lax.dot_general(jnp.ones((1, d), jnp.float32), c * c,
                           (((1,), (1,)), ((), ())),
                           precision=lax.Precision.HIGHEST)
    fc = lax.dot_general(ft, c, (((1,), (1,)), ((), ())),
                         precision=lax.Precision.DEFAULT)
    sq = (rown - 2.0 * fc) + coln
    labels = jnp.argmin(sq, axis=1).astype(jnp.int32)
    labels_ref[...] = labels.reshape(1, 1, tn)

    onehot = (labels[:, None] ==
              lax.broadcasted_iota(jnp.int32, (tn, k), 1)).astype(jnp.float32)
    counts_acc[...] += lax.dot_general(onehot, jnp.ones((tn, 128), jnp.float32),
                                       (((0,), (0,)), ((), ())),
                                       precision=lax.Precision.HIGHEST)

    @pl.when(i == nt - 1)
    def _write_counts():
        counts_ref[...] = counts_acc[...]


def _assign(features, sums8, counts_in):
    n, d = features.shape
    k = sums8.shape[2]
    nt = n // _TN
    labels3, counts = pl.pallas_call(
        _assign_body,
        grid=(nt,),
        in_specs=[
            pl.BlockSpec((_TN, d), lambda i: (i, 0)),
            pl.BlockSpec((_NRG, _NCG, k, _CW), lambda i: (0, 0, 0, 0)),
            pl.BlockSpec((k, 128), lambda i: (0, 0)),
        ],
        out_specs=[
            pl.BlockSpec((1, 1, _TN), lambda i: (i, 0, 0)),
            pl.BlockSpec((k, 128), lambda i: (0, 0)),
        ],
        out_shape=[
            jax.ShapeDtypeStruct((nt, 1, _TN), jnp.int32),
            jax.ShapeDtypeStruct((k, 128), jnp.float32),
        ],
        scratch_shapes=[pltpu.VMEM((k, d), jnp.float32),
                        pltpu.VMEM((k, 128), jnp.float32)],
    )(features, sums8, counts_in)
    return labels3.reshape(n), counts


def _segsum_body(feat_hbm, lab_hbm, zk_hbm, sums_out, labels_v, rows_v, acc_v):
    ci = lax.axis_index("c")
    si = lax.axis_index("s")
    w = ci * _NS + si
    rg = w // _NCG
    cg = w % _NCG
    pair = cg // 2
    half = cg % 2

    pltpu.sync_copy(zk_hbm, acc_v)
    pltpu.sync_copy(lab_hbm.at[pl.ds(rg * 16, 16)], labels_v)

    def chunk_body(ch, carry):
        row0 = rg * 2048 + ch * _CH
        pltpu.sync_copy(
            feat_hbm.at[pl.ds(row0, _CH), pl.ds(pair * 128, 128)], rows_v)

        def grp_body(g, c2):
            lbl_vec = labels_v[ch, pl.ds(g * 16, 16)]
            for r16 in range(16):
                base = lbl_vec[r16] * _CW
                r = g * 16 + r16
                for q in range(_CW // 16):
                    val = rows_v[r, pl.ds(half * _CW + q * 16, 16)]
                    plsc.addupdate(acc_v.at[pl.ds(base + q * 16, 16)], val)
            return c2

        return lax.fori_loop(0, _CH // 16, grp_body, carry)

    lax.fori_loop(0, 2048 // _CH, chunk_body, 0)
    pltpu.sync_copy(acc_v, sums_out.at[rg].at[cg])


def _segsum(features, labels):
    n, d = features.shape
    k = _K
    mesh = plsc.VectorSubcoreMesh(core_axis_name="c", subcore_axis_name="s",
                                  num_cores=_NC, num_subcores=_NS)
    fn = pl.kernel(
        _segsum_body,
        out_type=jax.ShapeDtypeStruct((_NRG, _NCG, k * _CW), jnp.float32),
        mesh=mesh,
        scratch_types=[
            pltpu.VMEM((16, _CH), jnp.int32),
            pltpu.VMEM((_CH, 128), jnp.float32),
            pltpu.VMEM((k * _CW,), jnp.float32),
        ],
    )
    lab2 = labels.reshape(n // _CH, _CH)
    zk = jnp.zeros((k * _CW,), jnp.float32)
    return fn(features, lab2, zk).reshape(_NRG, _NCG, k, _CW)


def _final_body(feat_ref, sums8_ref, counts_in_ref, labels_ref,
                ff_ref, dsum_ref, cent_v, acc_ref):
    i = pl.program_id(0)
    nt = pl.num_programs(0)
    ft = feat_ref[...]

    @pl.when(i == 0)
    def _update_centroids():
        s8 = sums8_ref[...]
        sp = (((s8[0] + s8[1]) + (s8[2] + s8[3])) +
              ((s8[4] + s8[5]) + (s8[6] + s8[7])))
        sums = jnp.concatenate([sp[j] for j in range(_NCG)], axis=1)
        counts = counts_in_ref[:, 0:1]
        cent_v[...] = jnp.where(counts > 0.0,
                                sums / jnp.maximum(counts, 1.0), 0.0)
        acc_ref[0, 0] = 0.0

    c = cent_v[...]
    tn, d = ft.shape
    k = c.shape[0]
    labels = labels_ref[0, 0, :]
    onehot = (labels[:, None] ==
              lax.broadcasted_iota(jnp.int32, (tn, k), 1)).astype(jnp.float32)
    ff = lax.dot_general(onehot, c, (((1,), (0,)), ((), ())),
                         precision=lax.Precision.HIGHEST)
    ff_ref[...] = ff

    diff = ft - ff
    acc_ref[0, 0] += jnp.sum(diff * diff)

    @pl.when(i == nt - 1)
    def _write():
        dsum_ref[0, 0] = acc_ref[0, 0]


def _finalize(features, sums8, counts_in, labels):
    n, d = features.shape
    k = sums8.shape[2]
    nt = n // _TN
    labels3 = labels.reshape(nt, 1, _TN)
    ff, dsum = pl.pallas_call(
        _final_body,
        grid=(nt,),
        in_specs=[
            pl.BlockSpec((_TN, d), lambda i: (i, 0)),
            pl.BlockSpec((_NRG, _NCG, k, _CW), lambda i: (0, 0, 0, 0)),
            pl.BlockSpec((k, 128), lambda i: (0, 0)),
            pl.BlockSpec((1, 1, _TN), lambda i: (i, 0, 0)),
        ],
        out_specs=[
            pl.BlockSpec((_TN, d), lambda i: (i, 0)),
            pl.BlockSpec(memory_space=pltpu.SMEM),
        ],
        out_shape=[
            jax.ShapeDtypeStruct((n, d), jnp.float32),
            jax.ShapeDtypeStruct((1, 1), jnp.float32),
        ],
        scratch_shapes=[pltpu.VMEM((k, d), jnp.float32),
                        pltpu.SMEM((1, 1), jnp.float32)],
    )(features, sums8, counts_in, labels3)
    return ff, dsum[0, 0]


def kernel(features, max_iters):
    n, d = features.shape
    perm = jax.random.permutation(jax.random.key(1), n)[:_K]
    cent0 = features[perm]
    cent0_4 = cent0.reshape(_K, _NCG, _CW).transpose(1, 0, 2)
    sums8_0 = jnp.zeros((_NRG, _NCG, _K, _CW), jnp.float32).at[0].set(cent0_4)
    counts_0 = jnp.ones((_K, 128), jnp.float32)
    labels0 = jnp.zeros((n,), jnp.int32)

    def body(_, carry):
        sums8, counts, _labels = carry
        labels, counts = _assign(features, sums8, counts)
        sums8 = _segsum(features, labels)
        return sums8, counts, labels

    sums8, counts, labels = lax.fori_loop(
        0, max_iters, body, (sums8_0, counts_0, labels0))
    ff, dsum = _finalize(features, sums8, counts, labels)
    differences = dsum / jnp.float32(n * d)
    return ff, labels, differences


# speedup vs baseline: 2.6559x; 2.6559x over previous
import jax
import jax.numpy as jnp
from jax import lax
from jax.experimental import pallas as pl
from jax.experimental.pallas import tpu as pltpu

_K = 1024
_TN = 1024


def _split3(x):
    hi = x.astype(jnp.bfloat16)
    r1 = x - hi.astype(jnp.float32)
    mid = r1.astype(jnp.bfloat16)
    lo = (r1 - mid.astype(jnp.float32)).astype(jnp.bfloat16)
    return hi, mid, lo


def _iter_body(feat_ref, cent_ref, labels_ref, newc_ref,
               sums_acc, counts_acc, coln_acc):
    i = pl.program_id(0)
    nt = pl.num_programs(0)
    ft = feat_ref[...]
    c = cent_ref[...]
    tn, d = ft.shape
    k = c.shape[0]

    @pl.when(i == 0)
    def _init():
        sums_acc[...] = jnp.zeros_like(sums_acc)
        counts_acc[...] = jnp.zeros_like(counts_acc)
        coln_acc[0:1, :] = lax.dot_general(
            jnp.ones((1, d), jnp.float32), c * c, (((1,), (1,)), ((), ())),
            precision=lax.Precision.HIGHEST)

    rown = jnp.sum(ft * ft, axis=1, keepdims=True)
    coln = coln_acc[0:1, :]
    fc = lax.dot_general(ft, c, (((1,), (1,)), ((), ())),
                         precision=lax.Precision.DEFAULT)
    sq = (rown - 2.0 * fc) + coln
    labels = jnp.argmin(sq, axis=1).astype(jnp.int32)
    labels_ref[...] = labels.reshape(1, 1, tn)

    onehot = (labels[:, None] ==
              lax.broadcasted_iota(jnp.int32, (tn, k), 1)).astype(jnp.bfloat16)
    hi, mid, lo = _split3(ft)
    dn = (((0,), (0,)), ((), ()))
    sums_acc[...] += (
        lax.dot_general(onehot, hi, dn, preferred_element_type=jnp.float32)
        + lax.dot_general(onehot, mid, dn, preferred_element_type=jnp.float32)
        + lax.dot_general(onehot, lo, dn, preferred_element_type=jnp.float32))
    counts_acc[...] += lax.dot_general(
        onehot, jnp.ones((tn, 128), jnp.bfloat16), dn,
        preferred_element_type=jnp.float32)

    @pl.when(i == nt - 1)
    def _update():
        counts = counts_acc[:, 0:1]
        sums = sums_acc[...]
        newc_ref[...] = jnp.where(counts > 0.0,
                                  sums / jnp.maximum(counts, 1.0), 0.0)


def _kmeans_iter(features, centroids):
    n, d = features.shape
    k = centroids.shape[0]
    nt = n // _TN
    labels3, newc = pl.pallas_call(
        _iter_body,
        grid=(nt,),
        in_specs=[
            pl.BlockSpec((_TN, d), lambda i: (i, 0)),
            pl.BlockSpec((k, d), lambda i: (0, 0)),
        ],
        out_specs=[
            pl.BlockSpec((1, 1, _TN), lambda i: (i, 0, 0)),
            pl.BlockSpec((k, d), lambda i: (0, 0)),
        ],
        out_shape=[
            jax.ShapeDtypeStruct((nt, 1, _TN), jnp.int32),
            jax.ShapeDtypeStruct((k, d), jnp.float32),
        ],
        scratch_shapes=[
            pltpu.VMEM((k, d), jnp.float32),
            pltpu.VMEM((k, 128), jnp.float32),
            pltpu.VMEM((8, k), jnp.float32),
        ],
    )(features, centroids)
    return labels3.reshape(n), newc


def _final_body(feat_ref, cent_ref, labels_ref, ff_ref, dsum_ref, acc_ref):
    i = pl.program_id(0)
    nt = pl.num_programs(0)
    ft = feat_ref[...]
    c = cent_ref[...]
    tn, d = ft.shape
    k = c.shape[0]
    labels = labels_ref[0, 0, :]

    onehot = (labels[:, None] ==
              lax.broadcasted_iota(jnp.int32, (tn, k), 1)).astype(jnp.bfloat16)
    chi, cmid, clo = _split3(c)
    dn = (((1,), (0,)), ((), ()))
    ff = (lax.dot_general(onehot, chi, dn, preferred_element_type=jnp.float32)
          + lax.dot_general(onehot, cmid, dn, preferred_element_type=jnp.float32)
          + lax.dot_general(onehot, clo, dn, preferred_element_type=jnp.float32))
    ff_ref[...] = ff

    diff = ft - ff
    part = jnp.sum(diff * diff)

    @pl.when(i == 0)
    def _init():
        acc_ref[0, 0] = 0.0

    acc_ref[0, 0] += part

    @pl.when(i == nt - 1)
    def _write():
        dsum_ref[0, 0] = acc_ref[0, 0]


def _finalize(features, centroids, labels):
    n, d = features.shape
    k = centroids.shape[0]
    nt = n // _TN
    labels3 = labels.reshape(nt, 1, _TN)
    ff, dsum = pl.pallas_call(
        _final_body,
        grid=(nt,),
        in_specs=[
            pl.BlockSpec((_TN, d), lambda i: (i, 0)),
            pl.BlockSpec((k, d), lambda i: (0, 0)),
            pl.BlockSpec((1, 1, _TN), lambda i: (i, 0, 0)),
        ],
        out_specs=[
            pl.BlockSpec((_TN, d), lambda i: (i, 0)),
            pl.BlockSpec(memory_space=pltpu.SMEM),
        ],
        out_shape=[
            jax.ShapeDtypeStruct((n, d), jnp.float32),
            jax.ShapeDtypeStruct((1, 1), jnp.float32),
        ],
        scratch_shapes=[pltpu.SMEM((1, 1), jnp.float32)],
    )(features, centroids, labels3)
    return ff, dsum[0, 0]


def kernel(features, max_iters):
    n, d = features.shape
    perm = jax.random.permutation(jax.random.key(1), n)[:_K]
    cent0 = features[perm]
    labels0 = jnp.zeros((n,), jnp.int32)

    def body(_, carry):
        cent, _labels = carry
        labels, newc = _kmeans_iter(features, cent)
        return newc, labels

    cent, labels = lax.fori_loop(0, max_iters, body, (cent0, labels0))
    ff, dsum = _finalize(features, cent, labels)
    differences = dsum / jnp.float32(n * d)
    return ff, labels, differences


# speedup vs baseline: 2.7630x; 1.0403x over previous
import jax
import jax.numpy as jnp
from jax import lax
from jax.experimental import pallas as pl
from jax.experimental.pallas import tpu as pltpu

_K = 1024
_TN = 1024


def _split3(x):
    hi = x.astype(jnp.bfloat16)
    r1 = x - hi.astype(jnp.float32)
    mid = r1.astype(jnp.bfloat16)
    lo = (r1 - mid.astype(jnp.float32)).astype(jnp.bfloat16)
    return hi, mid, lo


def _iter_body(feat_ref, hi_ref, mid_ref, lo_ref, cent_ref,
               labels_ref, newc_ref, sums_acc, counts_acc, coln_acc):
    i = pl.program_id(0)
    nt = pl.num_programs(0)
    ft = feat_ref[...]
    c = cent_ref[...]
    tn, d = ft.shape
    k = c.shape[0]

    @pl.when(i == 0)
    def _init():
        sums_acc[...] = jnp.zeros_like(sums_acc)
        counts_acc[...] = jnp.zeros_like(counts_acc)
        coln_acc[0:1, :] = lax.dot_general(
            jnp.ones((1, d), jnp.float32), c * c, (((1,), (1,)), ((), ())),
            precision=lax.Precision.HIGHEST)

    rown = jnp.sum(ft * ft, axis=1, keepdims=True)
    coln = coln_acc[0:1, :]
    fc = lax.dot_general(ft, c, (((1,), (1,)), ((), ())),
                         precision=lax.Precision.DEFAULT)
    sq = (rown - 2.0 * fc) + coln
    iota = lax.broadcasted_iota(jnp.int32, (tn, k), 1)
    m = jnp.min(sq, axis=1, keepdims=True)
    labels = jnp.min(jnp.where(sq == m, iota, k), axis=1)
    labels_ref[...] = labels.reshape(1, 1, tn)

    onehot = (labels[:, None] == iota).astype(jnp.bfloat16)
    dn = (((0,), (0,)), ((), ()))
    sums_acc[...] += (
        lax.dot_general(onehot, hi_ref[...], dn,
                        preferred_element_type=jnp.float32)
        + lax.dot_general(onehot, mid_ref[...], dn,
                          preferred_element_type=jnp.float32)
        + lax.dot_general(onehot, lo_ref[...], dn,
                          preferred_element_type=jnp.float32))
    counts_acc[...] += lax.dot_general(
        onehot, jnp.ones((tn, 128), jnp.bfloat16), dn,
        preferred_element_type=jnp.float32)

    @pl.when(i == nt - 1)
    def _update():
        counts = counts_acc[:, 0:1]
        sums = sums_acc[...]
        newc_ref[...] = jnp.where(counts > 0.0,
                                  sums / jnp.maximum(counts, 1.0), 0.0)


def _kmeans_iter(features, fhi, fmid, flo, centroids):
    n, d = features.shape
    k = centroids.shape[0]
    nt = n // _TN
    labels3, newc = pl.pallas_call(
        _iter_body,
        grid=(nt,),
        in_specs=[
            pl.BlockSpec((_TN, d), lambda i: (i, 0)),
            pl.BlockSpec((_TN, d), lambda i: (i, 0)),
            pl.BlockSpec((_TN, d), lambda i: (i, 0)),
            pl.BlockSpec((_TN, d), lambda i: (i, 0)),
            pl.BlockSpec((k, d), lambda i: (0, 0)),
        ],
        out_specs=[
            pl.BlockSpec((1, 1, _TN), lambda i: (i, 0, 0)),
            pl.BlockSpec((k, d), lambda i: (0, 0)),
        ],
        out_shape=[
            jax.ShapeDtypeStruct((nt, 1, _TN), jnp.int32),
            jax.ShapeDtypeStruct((k, d), jnp.float32),
        ],
        scratch_shapes=[
            pltpu.VMEM((k, d), jnp.float32),
            pltpu.VMEM((k, 128), jnp.float32),
            pltpu.VMEM((8, k), jnp.float32),
        ],
    )(features, fhi, fmid, flo, centroids)
    return labels3.reshape(n), newc


def _final_body(feat_ref, cent_ref, labels_ref, ff_ref, dsum_ref, acc_ref):
    i = pl.program_id(0)
    nt = pl.num_programs(0)
    ft = feat_ref[...]
    c = cent_ref[...]
    tn, d = ft.shape
    k = c.shape[0]
    labels = labels_ref[0, 0, :]

    onehot = (labels[:, None] ==
              lax.broadcasted_iota(jnp.int32, (tn, k), 1)).astype(jnp.bfloat16)
    chi, cmid, clo = _split3(c)
    dn = (((1,), (0,)), ((), ()))
    ff = (lax.dot_general(onehot, chi, dn, preferred_element_type=jnp.float32)
          + lax.dot_general(onehot, cmid, dn, preferred_element_type=jnp.float32)
          + lax.dot_general(onehot, clo, dn, preferred_element_type=jnp.float32))
    ff_ref[...] = ff

    diff = ft - ff
    part = jnp.sum(diff * diff)

    @pl.when(i == 0)
    def _init():
        acc_ref[0, 0] = 0.0

    acc_ref[0, 0] += part

    @pl.when(i == nt - 1)
    def _write():
        dsum_ref[0, 0] = acc_ref[0, 0]


def _finalize(features, centroids, labels):
    n, d = features.shape
    k = centroids.shape[0]
    nt = n // _TN
    labels3 = labels.reshape(nt, 1, _TN)
    ff, dsum = pl.pallas_call(
        _final_body,
        grid=(nt,),
        in_specs=[
            pl.BlockSpec((_TN, d), lambda i: (i, 0)),
            pl.BlockSpec((k, d), lambda i: (0, 0)),
            pl.BlockSpec((1, 1, _TN), lambda i: (i, 0, 0)),
        ],
        out_specs=[
            pl.BlockSpec((_TN, d), lambda i: (i, 0)),
            pl.BlockSpec(memory_space=pltpu.SMEM),
        ],
        out_shape=[
            jax.ShapeDtypeStruct((n, d), jnp.float32),
            jax.ShapeDtypeStruct((1, 1), jnp.float32),
        ],
        scratch_shapes=[pltpu.SMEM((1, 1), jnp.float32)],
    )(features, centroids, labels3)
    return ff, dsum[0, 0]


def kernel(features, max_iters):
    n, d = features.shape
    perm = jax.random.permutation(jax.random.key(1), n)[:_K]
    cent0 = features[perm]
    labels0 = jnp.zeros((n,), jnp.int32)
    fhi = features.astype(jnp.bfloat16)
    r1 = features - fhi.astype(jnp.float32)
    fmid = r1.astype(jnp.bfloat16)
    flo = (r1 - fmid.astype(jnp.float32)).astype(jnp.bfloat16)

    def body(_, carry):
        cent, _labels = carry
        labels, newc = _kmeans_iter(features, fhi, fmid, flo, cent)
        return newc, labels

    cent, labels = lax.fori_loop(0, max_iters, body, (cent0, labels0))
    ff, dsum = _finalize(features, cent, labels)
    differences = dsum / jnp.float32(n * d)
    return ff, labels, differences
